# baseline (device time: 28016 ns/iter reference)
import jax
import jax.numpy as jnp
from jax import lax
from jax.experimental import pallas as pl
from jax.experimental.pallas import tpu as pltpu

B, S, H_LOCAL, D = 4, 512, 8, 64
K = H_LOCAL * D
N = 1024
S_HALF = S // 2
R = 128
NSLOT = 2 * B
NC = 4


def kernel(O, Wo):
    Or = O.astype(jnp.bfloat16).reshape(B, S, K)
    Wb = Wo.astype(jnp.bfloat16)

    def body(o_ref, w_ref, out_ref, send_buf, recv_buf,
             y_send_sems, y_recv_sems, x_send_sems, x_recv_sems):
        my_x = lax.axis_index("x")
        my_y = lax.axis_index("y")
        peer_y = 1 - my_y
        peer_x = 1 - my_x

        barrier_sem = pltpu.get_barrier_semaphore()
        pl.semaphore_signal(
            barrier_sem, inc=1,
            device_id=(my_x, peer_y), device_id_type=pl.DeviceIdType.MESH,
        )
        pl.semaphore_signal(
            barrier_sem, inc=1,
            device_id=(peer_x, my_y), device_id_type=pl.DeviceIdType.MESH,
        )
        pl.semaphore_wait(barrier_sem, 2)

        w = w_ref[:]

        y_rdmas = []
        for j in range(NC):
            b = j // 2
            row0 = peer_y * S_HALF + (j % 2) * R
            ob = o_ref[pl.ds(2 * my_x + b, 1), pl.ds(row0, R), :].reshape(R, K)
            send_buf[j] = jnp.dot(
                ob, w, preferred_element_type=jnp.float32
            ).astype(jnp.bfloat16)
            rdma = pltpu.make_async_remote_copy(
                src_ref=send_buf.at[pl.ds(j, 1)],
                dst_ref=recv_buf.at[pl.ds(4 * my_x + j, 1)],
                send_sem=y_send_sems.at[j],
                recv_sem=y_recv_sems.at[j],
                device_id=(my_x, peer_y),
                device_id_type=pl.DeviceIdType.MESH,
            )
            rdma.start()
            y_rdmas.append(rdma)

        for b in range(B):
            ob = o_ref[b, pl.ds(my_y * S_HALF, S_HALF), :]
            acc = jnp.dot(ob, w, preferred_element_type=jnp.float32)
            out_ref[pl.ds(2 * b, 2)] = acc.reshape(2, R, N)

        x_rdmas = []
        for j in range(NC):
            s = 4 * my_x + j
            y_rdmas[j].wait_recv()
            fwd = pltpu.make_async_remote_copy(
                src_ref=recv_buf.at[pl.ds(s, 1)],
                dst_ref=recv_buf.at[pl.ds(s, 1)],
                send_sem=x_send_sems.at[j],
                recv_sem=x_recv_sems.at[j],
                device_id=(peer_x, my_y),
                device_id_type=pl.DeviceIdType.MESH,
            )
            fwd.start()
            x_rdmas.append(fwd)
            out_ref[pl.ds(s, 1)] = (
                out_ref[pl.ds(s, 1)]
                + recv_buf[pl.ds(s, 1)].astype(jnp.float32)
            )

        for j in range(NC):
            sx = 4 * peer_x + j
            x_rdmas[j].wait()
            out_ref[pl.ds(sx, 1)] = (
                out_ref[pl.ds(sx, 1)]
                + recv_buf[pl.ds(sx, 1)].astype(jnp.float32)
            )

        for j in range(NC):
            y_rdmas[j].wait_send()

    out = pl.pallas_call(
        body,
        out_shape=jax.ShapeDtypeStruct((NSLOT, R, N), jnp.float32),
        in_specs=[
            pl.BlockSpec(memory_space=pltpu.VMEM),
            pl.BlockSpec(memory_space=pltpu.VMEM),
        ],
        out_specs=pl.BlockSpec(memory_space=pltpu.VMEM),
        scratch_shapes=[
            pltpu.VMEM((NC, R, N), jnp.bfloat16),
            pltpu.VMEM((NSLOT, R, N), jnp.bfloat16),
            pltpu.SemaphoreType.DMA((NC,)),
            pltpu.SemaphoreType.DMA((NC,)),
            pltpu.SemaphoreType.DMA((NC,)),
            pltpu.SemaphoreType.DMA((NC,)),
        ],
        compiler_params=pltpu.CompilerParams(collective_id=0),
    )(Or, Wb)
    return out.reshape(B, S_HALF, N)


# device time: 25986 ns/iter; 1.0781x vs baseline; 1.0781x over previous
import jax
import jax.numpy as jnp
from jax import lax
from jax.experimental import pallas as pl
from jax.experimental.pallas import tpu as pltpu

B, S, H_LOCAL, D = 4, 512, 8, 64
K = H_LOCAL * D
N = 1024
S_HALF = S // 2
R = 64
NSLOT = 4 * B
NC = 8


def kernel(O, Wo):
    Or = O.reshape(B, S, K)

    def body(o_ref, w_ref, out_ref, send_buf, recv_buf,
             y_send_sems, y_recv_sems, x_send_sems, x_recv_sems):
        my_x = lax.axis_index("x")
        my_y = lax.axis_index("y")
        peer_y = 1 - my_y
        peer_x = 1 - my_x

        barrier_sem = pltpu.get_barrier_semaphore()
        pl.semaphore_signal(
            barrier_sem, inc=1,
            device_id=(my_x, peer_y), device_id_type=pl.DeviceIdType.MESH,
        )
        pl.semaphore_signal(
            barrier_sem, inc=1,
            device_id=(peer_x, my_y), device_id_type=pl.DeviceIdType.MESH,
        )
        pl.semaphore_wait(barrier_sem, 2)

        w = w_ref[:].astype(jnp.bfloat16)

        y_rdmas = []
        for j in range(NC):
            b = j // 4
            row0 = peer_y * S_HALF + (j % 4) * R
            ob = o_ref[
                pl.ds(2 * my_x + b, 1), pl.ds(row0, R), :
            ].reshape(R, K).astype(jnp.bfloat16)
            send_buf[j] = jnp.dot(
                ob, w, preferred_element_type=jnp.float32
            ).astype(jnp.bfloat16)
            rdma = pltpu.make_async_remote_copy(
                src_ref=send_buf.at[pl.ds(j, 1)],
                dst_ref=recv_buf.at[pl.ds(8 * my_x + j, 1)],
                send_sem=y_send_sems.at[j],
                recv_sem=y_recv_sems.at[j],
                device_id=(my_x, peer_y),
                device_id_type=pl.DeviceIdType.MESH,
            )
            rdma.start()
            y_rdmas.append(rdma)

        for b in range(B):
            ob = o_ref[b, pl.ds(my_y * S_HALF, S_HALF), :].astype(jnp.bfloat16)
            acc = jnp.dot(ob, w, preferred_element_type=jnp.float32)
            out_ref[pl.ds(4 * b, 4)] = acc.reshape(4, R, N)

        x_rdmas = []
        for j in range(NC):
            s = 8 * my_x + j
            y_rdmas[j].wait_recv()
            fwd = pltpu.make_async_remote_copy(
                src_ref=recv_buf.at[pl.ds(s, 1)],
                dst_ref=recv_buf.at[pl.ds(s, 1)],
                send_sem=x_send_sems.at[j],
                recv_sem=x_recv_sems.at[j],
                device_id=(peer_x, my_y),
                device_id_type=pl.DeviceIdType.MESH,
            )
            fwd.start()
            x_rdmas.append(fwd)
            out_ref[pl.ds(s, 1)] = (
                out_ref[pl.ds(s, 1)]
                + recv_buf[pl.ds(s, 1)].astype(jnp.float32)
            )

        for j in range(NC):
            sx = 8 * peer_x + j
            x_rdmas[j].wait()
            out_ref[pl.ds(sx, 1)] = (
                out_ref[pl.ds(sx, 1)]
                + recv_buf[pl.ds(sx, 1)].astype(jnp.float32)
            )

        for j in range(NC):
            y_rdmas[j].wait_send()

    out = pl.pallas_call(
        body,
        out_shape=jax.ShapeDtypeStruct((NSLOT, R, N), jnp.float32),
        in_specs=[
            pl.BlockSpec(memory_space=pltpu.VMEM),
            pl.BlockSpec(memory_space=pltpu.VMEM),
        ],
        out_specs=pl.BlockSpec(memory_space=pltpu.VMEM),
        scratch_shapes=[
            pltpu.VMEM((NC, R, N), jnp.bfloat16),
            pltpu.VMEM((NSLOT, R, N), jnp.bfloat16),
            pltpu.SemaphoreType.DMA((NC,)),
            pltpu.SemaphoreType.DMA((NC,)),
            pltpu.SemaphoreType.DMA((NC,)),
            pltpu.SemaphoreType.DMA((NC,)),
        ],
        compiler_params=pltpu.CompilerParams(collective_id=0),
    )(Or, Wo)
    return out.reshape(B, S_HALF, N)


# device time: 25976 ns/iter; 1.0785x vs baseline; 1.0004x over previous
import jax
import jax.numpy as jnp
from jax import lax
from jax.experimental import pallas as pl
from jax.experimental.pallas import tpu as pltpu

B, S, H_LOCAL, D = 4, 512, 8, 64
K = H_LOCAL * D
N = 1024
S_HALF = S // 2
R = 64
NSLOT = 4 * B
NC = 8


def kernel(O, Wo):
    Or = O.reshape(B, S, K)

    def body(o_ref, w_ref, out_ref, send_buf, recv_buf,
             y_send_sems, y_recv_sems, x_send_sems, x_recv_sems):
        my_x = lax.axis_index("x")
        my_y = lax.axis_index("y")
        peer_y = 1 - my_y
        peer_x = 1 - my_x

        barrier_sem = pltpu.get_barrier_semaphore()
        pl.semaphore_signal(
            barrier_sem, inc=1,
            device_id=(my_x, peer_y), device_id_type=pl.DeviceIdType.MESH,
        )
        pl.semaphore_signal(
            barrier_sem, inc=1,
            device_id=(peer_x, my_y), device_id_type=pl.DeviceIdType.MESH,
        )
        pl.semaphore_wait(barrier_sem, 2)

        w = w_ref[:].astype(jnp.bfloat16)

        y_rdmas = []
        for j in range(NC):
            b = j // 4
            row0 = peer_y * S_HALF + (j % 4) * R
            ob = o_ref[
                pl.ds(2 * my_x + b, 1), pl.ds(row0, R), :
            ].reshape(R, K).astype(jnp.bfloat16)
            send_buf[j] = jnp.dot(
                ob, w, preferred_element_type=jnp.float32
            ).astype(jnp.bfloat16)
            rdma = pltpu.make_async_remote_copy(
                src_ref=send_buf.at[pl.ds(j, 1)],
                dst_ref=recv_buf.at[pl.ds(8 * my_x + j, 1)],
                send_sem=y_send_sems.at[j],
                recv_sem=y_recv_sems.at[j],
                device_id=(my_x, peer_y),
                device_id_type=pl.DeviceIdType.MESH,
            )
            rdma.start()
            y_rdmas.append(rdma)

        for b in range(B):
            ob = o_ref[b, pl.ds(my_y * S_HALF, S_HALF), :].astype(jnp.bfloat16)
            out_ref[b] = jnp.dot(ob, w, preferred_element_type=jnp.float32)

        x_rdmas = []
        for j in range(NC):
            s = 8 * my_x + j
            y_rdmas[j].wait_recv()
            fwd = pltpu.make_async_remote_copy(
                src_ref=recv_buf.at[pl.ds(s, 1)],
                dst_ref=recv_buf.at[pl.ds(s, 1)],
                send_sem=x_send_sems.at[j],
                recv_sem=x_recv_sems.at[j],
                device_id=(peer_x, my_y),
                device_id_type=pl.DeviceIdType.MESH,
            )
            fwd.start()
            x_rdmas.append(fwd)
            ob_idx = (pl.ds(2 * my_x + j // 4, 1), pl.ds((j % 4) * R, R))
            out_ref[ob_idx] = (
                out_ref[ob_idx] + recv_buf[pl.ds(s, 1)].astype(jnp.float32)
            )

        for j in range(NC):
            sx = 8 * peer_x + j
            x_rdmas[j].wait()
            ob_idx = (pl.ds(2 * peer_x + j // 4, 1), pl.ds((j % 4) * R, R))
            out_ref[ob_idx] = (
                out_ref[ob_idx] + recv_buf[pl.ds(sx, 1)].astype(jnp.float32)
            )

        for j in range(NC):
            y_rdmas[j].wait_send()

    out = pl.pallas_call(
        body,
        out_shape=jax.ShapeDtypeStruct((B, S_HALF, N), jnp.float32),
        in_specs=[
            pl.BlockSpec(memory_space=pltpu.VMEM),
            pl.BlockSpec(memory_space=pltpu.VMEM),
        ],
        out_specs=pl.BlockSpec(memory_space=pltpu.VMEM),
        scratch_shapes=[
            pltpu.VMEM((NC, R, N), jnp.bfloat16),
            pltpu.VMEM((NSLOT, R, N), jnp.bfloat16),
            pltpu.SemaphoreType.DMA((NC,)),
            pltpu.SemaphoreType.DMA((NC,)),
            pltpu.SemaphoreType.DMA((NC,)),
            pltpu.SemaphoreType.DMA((NC,)),
        ],
        compiler_params=pltpu.CompilerParams(collective_id=0),
    )(Or, Wo)
    return out
